# baseline (speedup 1.0000x reference)
import functools

import jax
import jax.numpy as jnp
from jax import lax
from jax.experimental import pallas as pl
from jax.experimental.pallas import tpu as pltpu
from jax.experimental.pallas import tpu_sc as plsc

N = 10000
D = 256
H = 128
E = 160000
NC = 2
NS = 16
C = 256
CH = 40
EPT = C * CH
E_PAD = EPT * NS
NPAD = 112
NROW = N + NPAD
RPT = NROW // NS
FULLZ = RPT // C
REMZ = RPT - FULLZ * C
BN = 1000


def _sc_agg_body(with_cnt, *refs):
    if with_cnt:
        (hflat, sdp, zrows, z16, o16, agg, cnt,
         acc, cacc, sda, rows, ones, semg) = refs
    else:
        (hflat, sdp, zrows, agg,
         acc, sda, rows, semg) = refs
    cid = lax.axis_index("c")
    sid = lax.axis_index("s")
    base = sid * RPT

    pltpu.sync_copy(zrows, rows)
    for k in range(FULLZ):
        pltpu.sync_copy(rows, acc.at[pl.ds(base + k * C, C)])
    pltpu.sync_copy(rows.at[pl.ds(0, REMZ)],
                    acc.at[pl.ds(base + FULLZ * C, REMZ)])
    if with_cnt:
        pltpu.sync_copy(z16, ones)
        for k in range(FULLZ):
            pltpu.sync_copy(ones, cacc.at[pl.ds(base + k * C, C)])
        pltpu.sync_copy(ones.at[pl.ds(0, REMZ)],
                        cacc.at[pl.ds(base + FULLZ * C, REMZ)])
        pltpu.sync_copy(o16, ones)

    plsc.subcore_barrier()

    def chunk(j, carry):
        pltpu.sync_copy(sdp.at[cid, sid, j], sda)
        pltpu.async_copy(hflat.at[sda.at[0]], rows, semg).wait()
        pltpu.sync_copy(rows, acc.at[sda.at[1]], add=True)
        if with_cnt:
            @pl.when((j < CH // 2) == (cid == 0))
            def _():
                pltpu.sync_copy(ones, cacc.at[sda.at[1]], add=True)
        return carry

    lax.fori_loop(0, CH, chunk, 0)

    plsc.subcore_barrier()
    pltpu.sync_copy(acc.at[pl.ds(base, RPT)], agg.at[cid, pl.ds(base, RPT)])
    if with_cnt:
        pltpu.sync_copy(cacc.at[pl.ds(base, RPT)],
                        cnt.at[cid, pl.ds(base, RPT)])


def _make_sc_agg(with_cnt):
    mesh = plsc.VectorSubcoreMesh(core_axis_name="c", subcore_axis_name="s",
                                  num_cores=NC, num_subcores=NS)
    out_type = (jax.ShapeDtypeStruct((NC, NROW, H), jnp.float32),)
    scratch = [
        pltpu.VMEM_SHARED((NROW, H), jnp.float32),
    ]
    if with_cnt:
        out_type = out_type + (
            jax.ShapeDtypeStruct((NC, NROW, 16), jnp.float32),)
        scratch.append(pltpu.VMEM_SHARED((NROW, 16), jnp.float32))
    scratch += [
        pltpu.VMEM((2, C), jnp.int32),
        pltpu.VMEM((C, H), jnp.float32),
    ]
    if with_cnt:
        scratch.append(pltpu.VMEM((C, 16), jnp.float32))
    scratch.append(pltpu.SemaphoreType.DMA)
    return pl.kernel(functools.partial(_sc_agg_body, with_cnt),
                     out_type=out_type, mesh=mesh, scratch_types=scratch,
                     compiler_params=pltpu.CompilerParams(
                         use_tc_tiling_on_sc=False))


_sc_agg_l1 = _make_sc_agg(True)
_sc_agg_l2 = _make_sc_agg(False)


def _tc_self_body(h_ref, wr_ref, b_ref, o_ref):
    o_ref[...] = jnp.dot(h_ref[...], wr_ref[...],
                         preferred_element_type=jnp.float32) + b_ref[...]


def _tc_self(h, Wr, bl):
    return pl.pallas_call(
        _tc_self_body,
        grid=(N // BN,),
        in_specs=[
            pl.BlockSpec((BN, D), lambda i: (i, 0)),
            pl.BlockSpec((D, D), lambda i: (0, 0)),
            pl.BlockSpec((1, D), lambda i: (0, 0)),
        ],
        out_specs=pl.BlockSpec((BN, D), lambda i: (i, 0)),
        out_shape=jax.ShapeDtypeStruct((N, D), jnp.float32),
    )(h, Wr.T, bl.reshape(1, D))


def _tc_combine_body(split, a_ref, c_ref, s_ref, wla_ref, wlb_ref, *o_refs):
    r = 1.0 / jnp.maximum(c_ref[0, :, 0:1] + c_ref[1, :, 0:1], 1.0)
    acc = jnp.dot(a_ref[0] * r, wla_ref[...],
                  preferred_element_type=jnp.float32)
    acc += jnp.dot(a_ref[1] * r, wlb_ref[...],
                   preferred_element_type=jnp.float32)
    acc += s_ref[...]
    if split:
        h = jnp.maximum(acc, 0.0)
        o_refs[0][...] = h
        o_refs[1][0] = h[:, :H]
        o_refs[1][1] = h[:, H:]
    else:
        o_refs[0][...] = acc


def _tc_combine(agg, cnt, s, Wl, split):
    out_shape = [jax.ShapeDtypeStruct((N, D), jnp.float32)]
    out_specs = [pl.BlockSpec((BN, D), lambda i: (i, 0))]
    if split:
        out_shape.append(jax.ShapeDtypeStruct((NC, N, H), jnp.float32))
        out_specs.append(pl.BlockSpec((NC, BN, H), lambda i: (0, i, 0)))
    return pl.pallas_call(
        functools.partial(_tc_combine_body, split),
        grid=(N // BN,),
        in_specs=[
            pl.BlockSpec((NC, BN, H), lambda i: (0, i, 0)),
            pl.BlockSpec((NC, BN, 16), lambda i: (0, i, 0)),
            pl.BlockSpec((BN, D), lambda i: (i, 0)),
            pl.BlockSpec((H, D), lambda i: (0, 0)),
            pl.BlockSpec((H, D), lambda i: (0, 0)),
        ],
        out_specs=out_specs,
        out_shape=out_shape,
    )(agg, cnt, s, Wl[:, :H].T, Wl[:, H:].T)


def kernel(x, edge_index, W1l, b1l, W1r, W2l, b2l, W2r):
    src = edge_index[0].astype(jnp.int32)
    dst = edge_index[1].astype(jnp.int32)
    npad_e = E_PAD - E
    pad = jnp.arange(npad_e, dtype=jnp.int32)
    src_p = jnp.concatenate([src, pad % N])
    dst_p = jnp.concatenate([dst, N + pad % NPAD])
    dstp = jnp.broadcast_to(dst_p.reshape(1, NS, CH, C), (NC, NS, CH, C))
    srcp1 = ((2 * src_p)[None, :] +
             jnp.array([[0], [1]], jnp.int32)).reshape(NC, NS, CH, C)
    sdp1 = jnp.stack([srcp1, dstp], axis=3)
    srcp2 = (src_p[None, :] +
             jnp.array([[0], [N]], jnp.int32)).reshape(NC, NS, CH, C)
    sdp2 = jnp.stack([srcp2, dstp], axis=3)
    zrows = jnp.zeros((C, H), jnp.float32)
    z16 = jnp.zeros((C, 16), jnp.float32)
    o16 = jnp.ones((C, 16), jnp.float32)

    agg1, cnt = _sc_agg_l1(x.reshape(2 * N, H), sdp1, zrows, z16, o16)
    s1 = _tc_self(x, W1r, b1l)
    h1, h2 = _tc_combine(agg1, cnt, s1, W1l, split=True)
    (agg2,) = _sc_agg_l2(h2.reshape(2 * N, H), sdp2, zrows)
    s2 = _tc_self(h1, W2r, b2l)
    (out,) = _tc_combine(agg2, cnt, s2, W2l, split=False)
    return out


# speedup vs baseline: 1.0394x; 1.0394x over previous
import functools

import jax
import jax.numpy as jnp
from jax import lax
from jax.experimental import pallas as pl
from jax.experimental.pallas import tpu as pltpu
from jax.experimental.pallas import tpu_sc as plsc

N = 10000
D = 256
H = 128
E = 160000
NC = 2
NS = 16
C = 256
CH = 40
EPT = C * CH
E_PAD = EPT * NS
NPAD = 112
NROW = N + NPAD
RPT = NROW // NS
FULLZ = RPT // C
REMZ = RPT - FULLZ * C
BN = 1000


def _sc_agg_body(with_cnt, *refs):
    if with_cnt:
        (hflat, sdp, zrows, z16, o16, agg, cnt,
         acc, cacc, sda, rows, ones, semg) = refs
    else:
        (hflat, sdp, zrows, agg,
         acc, sda, rows, semg) = refs
    cid = lax.axis_index("c")
    sid = lax.axis_index("s")
    base = sid * RPT

    pltpu.sync_copy(zrows, rows)
    for k in range(FULLZ):
        pltpu.sync_copy(rows, acc.at[pl.ds(base + k * C, C)])
    pltpu.sync_copy(rows.at[pl.ds(0, REMZ)],
                    acc.at[pl.ds(base + FULLZ * C, REMZ)])
    if with_cnt:
        pltpu.sync_copy(z16, ones)
        for k in range(FULLZ):
            pltpu.sync_copy(ones, cacc.at[pl.ds(base + k * C, C)])
        pltpu.sync_copy(ones.at[pl.ds(0, REMZ)],
                        cacc.at[pl.ds(base + FULLZ * C, REMZ)])
        pltpu.sync_copy(o16, ones)

    plsc.subcore_barrier()

    def chunk(j, carry):
        pltpu.sync_copy(sdp.at[sid, j], sda)
        if with_cnt:
            for k in range(C // 16):
                sl = pl.ds(k * 16, 16)
                sda[0, sl] = sda[0, sl] * 2 + cid
            pltpu.async_copy(hflat.at[sda.at[0]], rows, semg).wait()
        else:
            pltpu.async_copy(hflat.at[cid].at[sda.at[0]], rows, semg).wait()
        pltpu.sync_copy(rows, acc.at[sda.at[1]], add=True)
        if with_cnt:
            @pl.when((j < CH // 2) == (cid == 0))
            def _():
                pltpu.sync_copy(ones, cacc.at[sda.at[1]], add=True)
        return carry

    lax.fori_loop(0, CH, chunk, 0)

    plsc.subcore_barrier()
    pltpu.sync_copy(acc.at[pl.ds(base, RPT)], agg.at[cid, pl.ds(base, RPT)])
    if with_cnt:
        pltpu.sync_copy(cacc.at[pl.ds(base, RPT)],
                        cnt.at[cid, pl.ds(base, RPT)])


def _make_sc_agg(with_cnt):
    mesh = plsc.VectorSubcoreMesh(core_axis_name="c", subcore_axis_name="s",
                                  num_cores=NC, num_subcores=NS)
    out_type = (jax.ShapeDtypeStruct((NC, NROW, H), jnp.float32),)
    scratch = [
        pltpu.VMEM_SHARED((NROW, H), jnp.float32),
    ]
    if with_cnt:
        out_type = out_type + (
            jax.ShapeDtypeStruct((NC, NROW, 16), jnp.float32),)
        scratch.append(pltpu.VMEM_SHARED((NROW, 16), jnp.float32))
    scratch += [
        pltpu.VMEM((2, C), jnp.int32),
        pltpu.VMEM((C, H), jnp.float32),
    ]
    if with_cnt:
        scratch.append(pltpu.VMEM((C, 16), jnp.float32))
    scratch.append(pltpu.SemaphoreType.DMA)
    return pl.kernel(functools.partial(_sc_agg_body, with_cnt),
                     out_type=out_type, mesh=mesh, scratch_types=scratch,
                     compiler_params=pltpu.CompilerParams(
                         use_tc_tiling_on_sc=False))


_sc_agg_l1 = _make_sc_agg(True)
_sc_agg_l2 = _make_sc_agg(False)


def _tc_self_body(h_ref, wr_ref, b_ref, o_ref):
    o_ref[...] = jnp.dot(h_ref[...], wr_ref[...],
                         preferred_element_type=jnp.float32) + b_ref[...]


def _tc_self(h, Wr, bl):
    return pl.pallas_call(
        _tc_self_body,
        grid=(N // BN,),
        in_specs=[
            pl.BlockSpec((BN, D), lambda i: (i, 0)),
            pl.BlockSpec((D, D), lambda i: (0, 0)),
            pl.BlockSpec((1, D), lambda i: (0, 0)),
        ],
        out_specs=pl.BlockSpec((BN, D), lambda i: (i, 0)),
        out_shape=jax.ShapeDtypeStruct((N, D), jnp.float32),
    )(h, Wr.T, bl.reshape(1, D))


def _tc_self2_body(h2_ref, wra_ref, wrb_ref, b_ref, o_ref):
    acc = jnp.dot(h2_ref[0], wra_ref[...], preferred_element_type=jnp.float32)
    acc += jnp.dot(h2_ref[1], wrb_ref[...], preferred_element_type=jnp.float32)
    o_ref[...] = acc + b_ref[...]


def _tc_self2(h2, Wr, bl):
    return pl.pallas_call(
        _tc_self2_body,
        grid=(N // BN,),
        in_specs=[
            pl.BlockSpec((NC, BN, H), lambda i: (0, i, 0)),
            pl.BlockSpec((H, D), lambda i: (0, 0)),
            pl.BlockSpec((H, D), lambda i: (0, 0)),
            pl.BlockSpec((1, D), lambda i: (0, 0)),
        ],
        out_specs=pl.BlockSpec((BN, D), lambda i: (i, 0)),
        out_shape=jax.ShapeDtypeStruct((N, D), jnp.float32),
    )(h2, Wr[:, :H].T, Wr[:, H:].T, bl.reshape(1, D))


def _tc_combine_body(split, a_ref, c_ref, s_ref, wla_ref, wlb_ref, o_ref):
    r = 1.0 / jnp.maximum(c_ref[0, :, 0:1] + c_ref[1, :, 0:1], 1.0)
    acc = jnp.dot(a_ref[0] * r, wla_ref[...],
                  preferred_element_type=jnp.float32)
    acc += jnp.dot(a_ref[1] * r, wlb_ref[...],
                   preferred_element_type=jnp.float32)
    acc += s_ref[...]
    if split:
        h = jnp.maximum(acc, 0.0)
        o_ref[0] = h[:, :H]
        o_ref[1] = h[:, H:]
    else:
        o_ref[...] = acc


def _tc_combine(agg, cnt, s, Wl, split):
    if split:
        out_shape = jax.ShapeDtypeStruct((NC, N, H), jnp.float32)
        out_specs = pl.BlockSpec((NC, BN, H), lambda i: (0, i, 0))
    else:
        out_shape = jax.ShapeDtypeStruct((N, D), jnp.float32)
        out_specs = pl.BlockSpec((BN, D), lambda i: (i, 0))
    return pl.pallas_call(
        functools.partial(_tc_combine_body, split),
        grid=(N // BN,),
        in_specs=[
            pl.BlockSpec((NC, BN, H), lambda i: (0, i, 0)),
            pl.BlockSpec((NC, BN, 16), lambda i: (0, i, 0)),
            pl.BlockSpec((BN, D), lambda i: (i, 0)),
            pl.BlockSpec((H, D), lambda i: (0, 0)),
            pl.BlockSpec((H, D), lambda i: (0, 0)),
        ],
        out_specs=out_specs,
        out_shape=out_shape,
    )(agg, cnt, s, Wl[:, :H].T, Wl[:, H:].T)


def kernel(x, edge_index, W1l, b1l, W1r, W2l, b2l, W2r):
    src = edge_index[0].astype(jnp.int32)
    dst = edge_index[1].astype(jnp.int32)
    npad_e = E_PAD - E
    pad = jnp.arange(npad_e, dtype=jnp.int32)
    src_p = jnp.concatenate([src, pad % N])
    dst_p = jnp.concatenate([dst, N + pad % NPAD])
    sdp = jnp.stack([src_p.reshape(NS, CH, C), dst_p.reshape(NS, CH, C)],
                    axis=2)
    zrows = jnp.zeros((C, H), jnp.float32)
    z16 = jnp.zeros((C, 16), jnp.float32)
    o16 = jnp.ones((C, 16), jnp.float32)

    agg1, cnt = _sc_agg_l1(x.reshape(2 * N, H), sdp, zrows, z16, o16)
    s1 = _tc_self(x, W1r, b1l)
    h2 = _tc_combine(agg1, cnt, s1, W1l, split=True)
    (agg2,) = _sc_agg_l2(h2, sdp, zrows)
    s2 = _tc_self2(h2, W2r, b2l)
    out = _tc_combine(agg2, cnt, s2, W2l, split=False)
    return out


# speedup vs baseline: 1.0427x; 1.0032x over previous
import functools

import jax
import jax.numpy as jnp
from jax import lax
from jax.experimental import pallas as pl
from jax.experimental.pallas import tpu as pltpu
from jax.experimental.pallas import tpu_sc as plsc

N = 10000
D = 256
H = 128
E = 160000
NC = 2
NS = 16
C = 256
CH = 40
EPT = C * CH
E_PAD = EPT * NS
NPAD = 112
NROW = N + NPAD
RPT = NROW // NS
FULLZ = RPT // C
REMZ = RPT - FULLZ * C
BN = 2000


def _sc_agg_body(with_cnt, *refs):
    if with_cnt:
        (hflat, sdp, zrows, z16, o16, agg, cnt,
         acc, cacc, sda, rows, ones, semg) = refs
    else:
        (hflat, sdp, zrows, agg,
         acc, sda, rows, semg) = refs
    cid = lax.axis_index("c")
    sid = lax.axis_index("s")
    base = sid * RPT

    pltpu.sync_copy(zrows, rows)
    for k in range(FULLZ):
        pltpu.sync_copy(rows, acc.at[pl.ds(base + k * C, C)])
    pltpu.sync_copy(rows.at[pl.ds(0, REMZ)],
                    acc.at[pl.ds(base + FULLZ * C, REMZ)])
    if with_cnt:
        pltpu.sync_copy(z16, ones)
        for k in range(FULLZ):
            pltpu.sync_copy(ones, cacc.at[pl.ds(base + k * C, C)])
        pltpu.sync_copy(ones.at[pl.ds(0, REMZ)],
                        cacc.at[pl.ds(base + FULLZ * C, REMZ)])
        pltpu.sync_copy(o16, ones)

    plsc.subcore_barrier()

    def chunk(j, carry):
        pltpu.sync_copy(sdp.at[sid, j], sda)
        if with_cnt:
            for k in range(C // 16):
                sl = pl.ds(k * 16, 16)
                sda[0, sl] = sda[0, sl] * 2 + cid
            pltpu.async_copy(hflat.at[sda.at[0]], rows, semg).wait()
        else:
            pltpu.async_copy(hflat.at[cid].at[sda.at[0]], rows, semg).wait()
        pltpu.sync_copy(rows, acc.at[sda.at[1]], add=True)
        if with_cnt:
            @pl.when((j < CH // 2) == (cid == 0))
            def _():
                pltpu.sync_copy(ones, cacc.at[sda.at[1]], add=True)
        return carry

    lax.fori_loop(0, CH, chunk, 0)

    plsc.subcore_barrier()
    pltpu.sync_copy(acc.at[pl.ds(base, RPT)], agg.at[cid, pl.ds(base, RPT)])
    if with_cnt:
        pltpu.sync_copy(cacc.at[pl.ds(base, RPT)],
                        cnt.at[cid, pl.ds(base, RPT)])


def _make_sc_agg(with_cnt):
    mesh = plsc.VectorSubcoreMesh(core_axis_name="c", subcore_axis_name="s",
                                  num_cores=NC, num_subcores=NS)
    out_type = (jax.ShapeDtypeStruct((NC, NROW, H), jnp.float32),)
    scratch = [
        pltpu.VMEM_SHARED((NROW, H), jnp.float32),
    ]
    if with_cnt:
        out_type = out_type + (
            jax.ShapeDtypeStruct((NC, NROW, 16), jnp.float32),)
        scratch.append(pltpu.VMEM_SHARED((NROW, 16), jnp.float32))
    scratch += [
        pltpu.VMEM((2, C), jnp.int32),
        pltpu.VMEM((C, H), jnp.float32),
    ]
    if with_cnt:
        scratch.append(pltpu.VMEM((C, 16), jnp.float32))
    scratch.append(pltpu.SemaphoreType.DMA)
    return pl.kernel(functools.partial(_sc_agg_body, with_cnt),
                     out_type=out_type, mesh=mesh, scratch_types=scratch,
                     compiler_params=pltpu.CompilerParams(
                         use_tc_tiling_on_sc=False))


_sc_agg_l1 = _make_sc_agg(True)
_sc_agg_l2 = _make_sc_agg(False)


def _tc_self_body(h_ref, wr_ref, b_ref, o_ref):
    o_ref[...] = jnp.dot(h_ref[...], wr_ref[...],
                         preferred_element_type=jnp.float32) + b_ref[...]


def _tc_self(h, Wr, bl):
    return pl.pallas_call(
        _tc_self_body,
        grid=(N // BN,),
        in_specs=[
            pl.BlockSpec((BN, D), lambda i: (i, 0)),
            pl.BlockSpec((D, D), lambda i: (0, 0)),
            pl.BlockSpec((1, D), lambda i: (0, 0)),
        ],
        out_specs=pl.BlockSpec((BN, D), lambda i: (i, 0)),
        out_shape=jax.ShapeDtypeStruct((N, D), jnp.float32),
    )(h, Wr.T, bl.reshape(1, D))


def _tc_self2_body(h2_ref, wra_ref, wrb_ref, b_ref, o_ref):
    acc = jnp.dot(h2_ref[0], wra_ref[...], preferred_element_type=jnp.float32)
    acc += jnp.dot(h2_ref[1], wrb_ref[...], preferred_element_type=jnp.float32)
    o_ref[...] = acc + b_ref[...]


def _tc_self2(h2, Wr, bl):
    return pl.pallas_call(
        _tc_self2_body,
        grid=(N // BN,),
        in_specs=[
            pl.BlockSpec((NC, BN, H), lambda i: (0, i, 0)),
            pl.BlockSpec((H, D), lambda i: (0, 0)),
            pl.BlockSpec((H, D), lambda i: (0, 0)),
            pl.BlockSpec((1, D), lambda i: (0, 0)),
        ],
        out_specs=pl.BlockSpec((BN, D), lambda i: (i, 0)),
        out_shape=jax.ShapeDtypeStruct((N, D), jnp.float32),
    )(h2, Wr[:, :H].T, Wr[:, H:].T, bl.reshape(1, D))


def _tc_combine_body(split, a_ref, c_ref, s_ref, wla_ref, wlb_ref, o_ref):
    r = 1.0 / jnp.maximum(c_ref[0, :, 0:1] + c_ref[1, :, 0:1], 1.0)
    acc = jnp.dot(a_ref[0] * r, wla_ref[...],
                  preferred_element_type=jnp.float32)
    acc += jnp.dot(a_ref[1] * r, wlb_ref[...],
                   preferred_element_type=jnp.float32)
    acc += s_ref[...]
    if split:
        h = jnp.maximum(acc, 0.0)
        o_ref[0] = h[:, :H]
        o_ref[1] = h[:, H:]
    else:
        o_ref[...] = acc


def _tc_combine(agg, cnt, s, Wl, split):
    if split:
        out_shape = jax.ShapeDtypeStruct((NC, N, H), jnp.float32)
        out_specs = pl.BlockSpec((NC, BN, H), lambda i: (0, i, 0))
    else:
        out_shape = jax.ShapeDtypeStruct((N, D), jnp.float32)
        out_specs = pl.BlockSpec((BN, D), lambda i: (i, 0))
    return pl.pallas_call(
        functools.partial(_tc_combine_body, split),
        grid=(N // BN,),
        in_specs=[
            pl.BlockSpec((NC, BN, H), lambda i: (0, i, 0)),
            pl.BlockSpec((NC, BN, 16), lambda i: (0, i, 0)),
            pl.BlockSpec((BN, D), lambda i: (i, 0)),
            pl.BlockSpec((H, D), lambda i: (0, 0)),
            pl.BlockSpec((H, D), lambda i: (0, 0)),
        ],
        out_specs=out_specs,
        out_shape=out_shape,
    )(agg, cnt, s, Wl[:, :H].T, Wl[:, H:].T)


def kernel(x, edge_index, W1l, b1l, W1r, W2l, b2l, W2r):
    src = edge_index[0].astype(jnp.int32)
    dst = edge_index[1].astype(jnp.int32)
    npad_e = E_PAD - E
    pad = jnp.arange(npad_e, dtype=jnp.int32)
    src_p = jnp.concatenate([src, pad])
    dst_p = jnp.concatenate([dst, N + (pad & 63)])
    sdp = jnp.stack([src_p.reshape(NS, CH, C), dst_p.reshape(NS, CH, C)],
                    axis=2)
    zrows = jnp.zeros((C, H), jnp.float32)
    z16 = jnp.zeros((C, 16), jnp.float32)
    o16 = jnp.ones((C, 16), jnp.float32)

    agg1, cnt = _sc_agg_l1(x.reshape(2 * N, H), sdp, zrows, z16, o16)
    s1 = _tc_self(x, W1r, b1l)
    h2 = _tc_combine(agg1, cnt, s1, W1l, split=True)
    (agg2,) = _sc_agg_l2(h2, sdp, zrows)
    s2 = _tc_self2(h2, W2r, b2l)
    out = _tc_combine(agg2, cnt, s2, W2l, split=False)
    return out
